# baseline (speedup 1.0000x reference)
import functools

import jax
import jax.numpy as jnp
from jax.experimental import pallas as pl
from jax.experimental.pallas import tpu as pltpu

_N = 5000
_IOU_THRESHOLD = 0.6
_SCORE_THRESHOLD = 0.05
_B = 128
_T = 40
_NP = _B * _T


def _nms_body(boxes_ref, x1c_ref, y1c_ref, x2c_ref, y2c_ref, keep0_ref,
              out_ref, over_scratch):
    out_ref[...] = keep0_ref[...]
    ri = jax.lax.broadcasted_iota(jnp.int32, (_B, _B), 0)
    ci = jax.lax.broadcasted_iota(jnp.int32, (_B, _B), 1)
    tri = (ci > ri).astype(jnp.float32)
    eye = (ci == ri).astype(jnp.float32)

    def block_body(bi, carry):
        base = bi * _B
        blk = boxes_ref[pl.ds(base, _B), :]
        x1r = blk[:, 0:1]
        y1r = blk[:, 1:2]
        x2r = blk[:, 2:3]
        y2r = blk[:, 3:4]
        area_r = (x2r - x1r) * (y2r - y1r)

        def over_tile(cb):
            x1c = x1c_ref[pl.ds(cb, 1), :]
            y1c = y1c_ref[pl.ds(cb, 1), :]
            x2c = x2c_ref[pl.ds(cb, 1), :]
            y2c = y2c_ref[pl.ds(cb, 1), :]
            ltx = jnp.maximum(x1r, x1c)
            lty = jnp.maximum(y1r, y1c)
            rbx = jnp.minimum(x2r, x2c)
            rby = jnp.minimum(y2r, y2c)
            w = jnp.maximum(rbx - ltx, 0.0)
            h = jnp.maximum(rby - lty, 0.0)
            inter = w * h
            area_c = (x2c - x1c) * (y2c - y1c)
            union = area_r + area_c - inter
            iou = inter / jnp.maximum(union, 1e-9)
            return (iou > _IOU_THRESHOLD).astype(jnp.float32)

        over_scratch[...] = over_tile(bi) * tri
        keep_row = out_ref[pl.ds(bi, 1), :]

        def jbody(j, kr):
            r = over_scratch[pl.ds(j, 1), :]
            kj = pltpu.roll(kr, -j, axis=1)[0:1, 0:1]
            return kr * (1.0 - r * kj)

        keep_row = jax.lax.fori_loop(0, _B, jbody, keep_row, unroll=8)
        out_ref[pl.ds(bi, 1), :] = keep_row

        keep_col = jnp.sum(eye * keep_row, axis=1, keepdims=True)

        def cross(cb, c2):
            ov = over_tile(cb)
            sup = jnp.max(ov * keep_col, axis=0, keepdims=True)
            out_ref[pl.ds(cb, 1), :] = out_ref[pl.ds(cb, 1), :] * (1.0 - sup)
            return c2

        jax.lax.fori_loop(bi + 1, _T, cross, 0)
        return carry

    jax.lax.fori_loop(0, _T, block_body, 0)


@functools.partial(jax.jit, static_argnames=("interpret",))
def _nms_pallas(boxes_p, x1c, y1c, x2c, y2c, keep0, interpret=False):
    return pl.pallas_call(
        _nms_body,
        out_shape=jax.ShapeDtypeStruct((_T, _B), jnp.float32),
        scratch_shapes=[pltpu.VMEM((_B, _B), jnp.float32)],
        interpret=interpret,
    )(boxes_p, x1c, y1c, x2c, y2c, keep0)


def _run(boxes, scores, interpret=False):
    valid = scores > _SCORE_THRESHOLD
    eff = jnp.where(valid, scores, -1.0)
    order = jnp.argsort(-eff)
    b = boxes[order]
    s = eff[order]
    pad = _NP - _N
    b_p = jnp.pad(b, ((0, pad), (0, 0)))
    s_p = jnp.pad(s, (0, pad), constant_values=-1.0)
    keep0 = (s_p > 0.0).astype(jnp.float32).reshape(_T, _B)
    x1c = b_p[:, 0].reshape(_T, _B)
    y1c = b_p[:, 1].reshape(_T, _B)
    x2c = b_p[:, 2].reshape(_T, _B)
    y2c = b_p[:, 3].reshape(_T, _B)
    keep = _nms_pallas(b_p, x1c, y1c, x2c, y2c, keep0, interpret=interpret)
    keep_s = keep.reshape(_NP)[:_N] > 0.0
    kept_scores_sorted = jnp.maximum(s * keep_s.astype(jnp.float32), 0.0)
    out_scores = jnp.zeros((_N,), jnp.float32).at[order].set(kept_scores_sorted)
    keep_mask = jnp.zeros((_N,), bool).at[order].set(keep_s)
    return out_scores, keep_mask


def kernel(boxes, scores):
    return _run(boxes, scores)


# speedup vs baseline: 1.0264x; 1.0264x over previous
import functools

import jax
import jax.numpy as jnp
from jax.experimental import pallas as pl
from jax.experimental.pallas import tpu as pltpu

_N = 5000
_IOU_THRESHOLD = 0.6
_SCORE_THRESHOLD = 0.05
_B = 128
_T = 40
_NP = _B * _T


def _nms_body(boxes_ref, x1c_ref, y1c_ref, x2c_ref, y2c_ref, keep0_ref,
              out_ref, over_scratch):
    out_ref[...] = keep0_ref[...]
    ri = jax.lax.broadcasted_iota(jnp.int32, (_B, _B), 0)
    ci = jax.lax.broadcasted_iota(jnp.int32, (_B, _B), 1)
    tri = (ci > ri).astype(jnp.float32)
    eye = (ci == ri).astype(jnp.float32)

    def block_body(bi, carry):
        base = bi * _B
        blk = boxes_ref[pl.ds(base, _B), :]
        x1r = blk[:, 0:1]
        y1r = blk[:, 1:2]
        x2r = blk[:, 2:3]
        y2r = blk[:, 3:4]
        area_r = (x2r - x1r) * (y2r - y1r)

        def over_tile(cb):
            x1c = x1c_ref[pl.ds(cb, 1), :]
            y1c = y1c_ref[pl.ds(cb, 1), :]
            x2c = x2c_ref[pl.ds(cb, 1), :]
            y2c = y2c_ref[pl.ds(cb, 1), :]
            ltx = jnp.maximum(x1r, x1c)
            lty = jnp.maximum(y1r, y1c)
            rbx = jnp.minimum(x2r, x2c)
            rby = jnp.minimum(y2r, y2c)
            w = jnp.maximum(rbx - ltx, 0.0)
            h = jnp.maximum(rby - lty, 0.0)
            inter = w * h
            area_c = (x2c - x1c) * (y2c - y1c)
            union = area_r + area_c - inter
            iou = inter / jnp.maximum(union, 1e-9)
            return (iou > _IOU_THRESHOLD).astype(jnp.float32)

        over_scratch[...] = over_tile(bi) * tri
        keep_row = out_ref[pl.ds(bi, 1), :]

        def jbody(j, kr):
            r = over_scratch[pl.ds(j, 1), :]
            kj = pltpu.roll(kr, -j, axis=1)[0:1, 0:1]
            return kr * (1.0 - r * kj)

        keep_row = jax.lax.fori_loop(0, _B, jbody, keep_row)
        out_ref[pl.ds(bi, 1), :] = keep_row

        keep_col = jnp.sum(eye * keep_row, axis=1, keepdims=True)

        def cross(cb, c2):
            ov = over_tile(cb)
            sup = jnp.max(ov * keep_col, axis=0, keepdims=True)
            out_ref[pl.ds(cb, 1), :] = out_ref[pl.ds(cb, 1), :] * (1.0 - sup)
            return c2

        jax.lax.fori_loop(bi + 1, _T, cross, 0)
        return carry

    jax.lax.fori_loop(0, _T, block_body, 0)


@functools.partial(jax.jit, static_argnames=("interpret",))
def _nms_pallas(boxes_p, x1c, y1c, x2c, y2c, keep0, interpret=False):
    return pl.pallas_call(
        _nms_body,
        out_shape=jax.ShapeDtypeStruct((_T, _B), jnp.float32),
        scratch_shapes=[pltpu.VMEM((_B, _B), jnp.float32)],
        interpret=interpret,
    )(boxes_p, x1c, y1c, x2c, y2c, keep0)


def _run(boxes, scores, interpret=False):
    valid = scores > _SCORE_THRESHOLD
    eff = jnp.where(valid, scores, -1.0)
    order = jnp.argsort(-eff)
    b = boxes[order]
    s = eff[order]
    pad = _NP - _N
    b_p = jnp.pad(b, ((0, pad), (0, 0)))
    s_p = jnp.pad(s, (0, pad), constant_values=-1.0)
    keep0 = (s_p > 0.0).astype(jnp.float32).reshape(_T, _B)
    x1c = b_p[:, 0].reshape(_T, _B)
    y1c = b_p[:, 1].reshape(_T, _B)
    x2c = b_p[:, 2].reshape(_T, _B)
    y2c = b_p[:, 3].reshape(_T, _B)
    keep = _nms_pallas(b_p, x1c, y1c, x2c, y2c, keep0, interpret=interpret)
    keep_s = keep.reshape(_NP)[:_N] > 0.0
    kept_scores_sorted = jnp.maximum(s * keep_s.astype(jnp.float32), 0.0)
    out_scores = jnp.zeros((_N,), jnp.float32).at[order].set(kept_scores_sorted)
    keep_mask = jnp.zeros((_N,), bool).at[order].set(keep_s)
    return out_scores, keep_mask


def kernel(boxes, scores):
    return _run(boxes, scores)


# speedup vs baseline: 1.1232x; 1.0943x over previous
import functools

import jax
import jax.numpy as jnp
from jax.experimental import pallas as pl
from jax.experimental.pallas import tpu as pltpu

_N = 5000
_IOU_THRESHOLD = 0.6
_SCORE_THRESHOLD = 0.05
_B = 128
_T = 40
_NP = _B * _T


def _nms_body(boxes_ref, x1c_ref, y1c_ref, x2c_ref, y2c_ref, keep0_ref,
              out_ref, over_scratch):
    out_ref[...] = keep0_ref[...]
    ri = jax.lax.broadcasted_iota(jnp.int32, (_B, _B), 0)
    ci = jax.lax.broadcasted_iota(jnp.int32, (_B, _B), 1)
    tri = (ci > ri).astype(jnp.float32)
    eye = (ci == ri).astype(jnp.float32)

    def block_body(bi, carry):
        base = bi * _B
        blk = boxes_ref[pl.ds(base, _B), :]
        x1r = blk[:, 0:1]
        y1r = blk[:, 1:2]
        x2r = blk[:, 2:3]
        y2r = blk[:, 3:4]
        area_r = (x2r - x1r) * (y2r - y1r)

        def over_tile(cb):
            x1c = x1c_ref[pl.ds(cb, 1), :]
            y1c = y1c_ref[pl.ds(cb, 1), :]
            x2c = x2c_ref[pl.ds(cb, 1), :]
            y2c = y2c_ref[pl.ds(cb, 1), :]
            ltx = jnp.maximum(x1r, x1c)
            lty = jnp.maximum(y1r, y1c)
            rbx = jnp.minimum(x2r, x2c)
            rby = jnp.minimum(y2r, y2c)
            w = jnp.maximum(rbx - ltx, 0.0)
            h = jnp.maximum(rby - lty, 0.0)
            inter = w * h
            area_c = (x2c - x1c) * (y2c - y1c)
            union = area_r + area_c - inter
            iou = inter / jnp.maximum(union, 1e-9)
            return (iou > _IOU_THRESHOLD).astype(jnp.float32)

        over_scratch[...] = over_tile(bi) * tri
        keep_row = out_ref[pl.ds(bi, 1), :]

        lane = jax.lax.broadcasted_iota(jnp.int32, (1, _B), 1)

        def jbody(j, kr):
            r = over_scratch[pl.ds(j, 1), :]
            kj = jnp.max(jnp.where(lane == j, kr, 0.0))
            return kr * (1.0 - r * kj)

        keep_row = jax.lax.fori_loop(0, _B, jbody, keep_row, unroll=8)
        out_ref[pl.ds(bi, 1), :] = keep_row

        keep_col = jnp.sum(eye * keep_row, axis=1, keepdims=True)

        def cross(cb, c2):
            ov = over_tile(cb)
            sup = jnp.max(ov * keep_col, axis=0, keepdims=True)
            out_ref[pl.ds(cb, 1), :] = out_ref[pl.ds(cb, 1), :] * (1.0 - sup)
            return c2

        jax.lax.fori_loop(bi + 1, _T, cross, 0)
        return carry

    jax.lax.fori_loop(0, _T, block_body, 0)


@functools.partial(jax.jit, static_argnames=("interpret",))
def _nms_pallas(boxes_p, x1c, y1c, x2c, y2c, keep0, interpret=False):
    return pl.pallas_call(
        _nms_body,
        out_shape=jax.ShapeDtypeStruct((_T, _B), jnp.float32),
        scratch_shapes=[pltpu.VMEM((_B, _B), jnp.float32)],
        interpret=interpret,
    )(boxes_p, x1c, y1c, x2c, y2c, keep0)


def _run(boxes, scores, interpret=False):
    valid = scores > _SCORE_THRESHOLD
    eff = jnp.where(valid, scores, -1.0)
    order = jnp.argsort(-eff)
    b = boxes[order]
    s = eff[order]
    pad = _NP - _N
    b_p = jnp.pad(b, ((0, pad), (0, 0)))
    s_p = jnp.pad(s, (0, pad), constant_values=-1.0)
    keep0 = (s_p > 0.0).astype(jnp.float32).reshape(_T, _B)
    x1c = b_p[:, 0].reshape(_T, _B)
    y1c = b_p[:, 1].reshape(_T, _B)
    x2c = b_p[:, 2].reshape(_T, _B)
    y2c = b_p[:, 3].reshape(_T, _B)
    keep = _nms_pallas(b_p, x1c, y1c, x2c, y2c, keep0, interpret=interpret)
    keep_s = keep.reshape(_NP)[:_N] > 0.0
    kept_scores_sorted = jnp.maximum(s * keep_s.astype(jnp.float32), 0.0)
    out_scores = jnp.zeros((_N,), jnp.float32).at[order].set(kept_scores_sorted)
    keep_mask = jnp.zeros((_N,), bool).at[order].set(keep_s)
    return out_scores, keep_mask


def kernel(boxes, scores):
    return _run(boxes, scores)


# speedup vs baseline: 2.2200x; 1.9765x over previous
import functools

import jax
import jax.numpy as jnp
import numpy as np
from jax.experimental import pallas as pl
from jax.experimental.pallas import tpu as pltpu

_N = 5000
_IOU_THRESHOLD = 0.6
_SCORE_THRESHOLD = 0.05
_B = 128
_T = 40
_NP = _B * _T
_W = 8


def _nms_body(keep0_ref, boxes_ref, x1c_ref, y1c_ref, x2c_ref, y2c_ref,
              wpack_ref, out_ref, ks, pk_vmem, pk_smem, sem):
    bi = pl.program_id(0)

    @pl.when(bi == 0)
    def _():
        ks[...] = keep0_ref[...]

    base = bi * _B
    blk = boxes_ref[pl.ds(base, _B), :]
    x1r = blk[:, 0:1]
    y1r = blk[:, 1:2]
    x2r = blk[:, 2:3]
    y2r = blk[:, 3:4]
    area_r = (x2r - x1r) * (y2r - y1r)

    def over_tile(cb):
        x1c = x1c_ref[pl.ds(cb, 1), :]
        y1c = y1c_ref[pl.ds(cb, 1), :]
        x2c = x2c_ref[pl.ds(cb, 1), :]
        y2c = y2c_ref[pl.ds(cb, 1), :]
        ltx = jnp.maximum(x1r, x1c)
        lty = jnp.maximum(y1r, y1c)
        rbx = jnp.minimum(x2r, x2c)
        rby = jnp.minimum(y2r, y2c)
        w = jnp.maximum(rbx - ltx, 0.0)
        h = jnp.maximum(rby - lty, 0.0)
        inter = w * h
        area_c = (x2c - x1c) * (y2c - y1c)
        union = area_r + area_c - inter
        iou = inter / jnp.maximum(union, 1e-9)
        return (iou > _IOU_THRESHOLD).astype(jnp.float32)

    ri = jax.lax.broadcasted_iota(jnp.int32, (_B, _B), 0)
    ci = jax.lax.broadcasted_iota(jnp.int32, (_B, _B), 1)
    tri = (ci > ri).astype(jnp.float32)
    ov_bb = over_tile(bi) * tri
    wp = wpack_ref[...]
    dn = (((1,), (0,)), ((), ()))
    packed_rows = jax.lax.dot_general(
        ov_bb.astype(jnp.bfloat16), wp, dn,
        preferred_element_type=jnp.float32).astype(jnp.int32)
    kr = ks[pl.ds(bi, 1), :]
    packed_kr = jax.lax.dot_general(
        kr.astype(jnp.bfloat16), wp, dn,
        preferred_element_type=jnp.float32).astype(jnp.int32)
    pk_vmem[pl.ds(0, _B), :] = packed_rows
    pk_vmem[pl.ds(_B, 1), :] = packed_kr
    copy = pltpu.make_async_copy(pk_vmem, pk_smem, sem)
    copy.start()
    copy.wait()

    words = tuple(pk_smem[_B, w] for w in range(_W))

    def make_inner(w):
        def inner(j2, ws):
            kj = (ws[w] >> j2) & 1
            m = -kj
            j = w * 16 + j2
            new = list(ws)
            for k in range(w, _W):
                new[k] = ws[k] & ~(pk_smem[j, k] & m)
            return tuple(new)
        return inner

    for w in range(_W):
        words = jax.lax.fori_loop(0, 16, make_inner(w), words)

    lanei = jax.lax.broadcasted_iota(jnp.int32, (1, _B), 1)
    widx = lanei >> 4
    bidx = lanei & 15
    wsel = jnp.zeros((1, _B), jnp.int32)
    for w in range(_W):
        wsel = jnp.where(widx == w, words[w], wsel)
    kr_new = ((wsel >> bidx) & 1).astype(jnp.float32)
    ks[pl.ds(bi, 1), :] = kr_new
    out_ref[...] = kr_new.reshape(1, 1, _B)

    krb = kr_new.astype(jnp.bfloat16)

    def cross(cb, c2):
        ov = over_tile(cb).astype(jnp.bfloat16)
        cnt = jax.lax.dot_general(krb, ov, dn,
                                  preferred_element_type=jnp.float32)
        kcb = ks[pl.ds(cb, 1), :]
        ks[pl.ds(cb, 1), :] = jnp.where(cnt > 0.0, 0.0, kcb)
        return c2

    jax.lax.fori_loop(bi + 1, _T, cross, 0)


@functools.partial(jax.jit, static_argnames=("interpret",))
def _nms_pallas(keep0, boxes_p, x1c, y1c, x2c, y2c, wpack, interpret=False):
    full2 = lambda shape: pl.BlockSpec(shape, lambda bi: (0, 0))
    return pl.pallas_call(
        _nms_body,
        grid=(_T,),
        in_specs=[
            full2((_T, _B)),
            full2((_NP, 4)),
            full2((_T, _B)), full2((_T, _B)), full2((_T, _B)), full2((_T, _B)),
            full2((_B, _W)),
        ],
        out_specs=pl.BlockSpec((1, 1, _B), lambda bi: (bi, 0, 0)),
        out_shape=jax.ShapeDtypeStruct((_T, 1, _B), jnp.float32),
        scratch_shapes=[pltpu.VMEM((_T, _B), jnp.float32),
                        pltpu.VMEM((_B + 1, _W), jnp.int32),
                        pltpu.SMEM((_B + 1, _W), jnp.int32),
                        pltpu.SemaphoreType.DMA],
        interpret=interpret,
    )(keep0, boxes_p, x1c, y1c, x2c, y2c, wpack)


_l = np.arange(_B)
_wpack_np = np.zeros((_B, _W), np.float32)
_wpack_np[_l, _l // 16] = 2.0 ** (_l % 16)


def _run(boxes, scores, interpret=False):
    valid = scores > _SCORE_THRESHOLD
    eff = jnp.where(valid, scores, -1.0)
    order = jnp.argsort(-eff)
    b = boxes[order]
    s = eff[order]
    pad = _NP - _N
    b_p = jnp.pad(b, ((0, pad), (0, 0)))
    s_p = jnp.pad(s, (0, pad), constant_values=-1.0)
    keep0 = (s_p > 0.0).astype(jnp.float32).reshape(_T, _B)
    x1c = b_p[:, 0].reshape(_T, _B)
    y1c = b_p[:, 1].reshape(_T, _B)
    x2c = b_p[:, 2].reshape(_T, _B)
    y2c = b_p[:, 3].reshape(_T, _B)
    wpack = jnp.asarray(_wpack_np, jnp.bfloat16)
    keep = _nms_pallas(keep0, b_p, x1c, y1c, x2c, y2c, wpack,
                       interpret=interpret)
    keep_s = keep.reshape(_NP)[:_N] > 0.0
    kept_scores_sorted = jnp.maximum(s * keep_s.astype(jnp.float32), 0.0)
    out_scores = jnp.zeros((_N,), jnp.float32).at[order].set(kept_scores_sorted)
    keep_mask = jnp.zeros((_N,), bool).at[order].set(keep_s)
    return out_scores, keep_mask


def kernel(boxes, scores):
    return _run(boxes, scores)


# speedup vs baseline: 2.9461x; 1.3270x over previous
import functools

import jax
import jax.numpy as jnp
import numpy as np
from jax.experimental import pallas as pl
from jax.experimental.pallas import tpu as pltpu

_N = 5000
_IOU_THRESHOLD = 0.6
_SCORE_THRESHOLD = 0.05
_B = 128
_T = 40
_NP = _B * _T
_W = 8


def _nms_body(keep0_ref, boxes_ref, x1c_ref, y1c_ref, x2c_ref, y2c_ref,
              wpack_ref, out_ref, ks, pk_vmem, pk_smem, sem):
    bi = pl.program_id(0)

    @pl.when(bi == 0)
    def _():
        ks[...] = keep0_ref[...]

    base = bi * _B
    blk = boxes_ref[pl.ds(base, _B), :]
    x1r = blk[:, 0:1]
    y1r = blk[:, 1:2]
    x2r = blk[:, 2:3]
    y2r = blk[:, 3:4]
    area_r = (x2r - x1r) * (y2r - y1r)

    def over_tile(cb):
        x1c = x1c_ref[pl.ds(cb, 1), :]
        y1c = y1c_ref[pl.ds(cb, 1), :]
        x2c = x2c_ref[pl.ds(cb, 1), :]
        y2c = y2c_ref[pl.ds(cb, 1), :]
        ltx = jnp.maximum(x1r, x1c)
        lty = jnp.maximum(y1r, y1c)
        rbx = jnp.minimum(x2r, x2c)
        rby = jnp.minimum(y2r, y2c)
        w = jnp.maximum(rbx - ltx, 0.0)
        h = jnp.maximum(rby - lty, 0.0)
        inter = w * h
        area_c = (x2c - x1c) * (y2c - y1c)
        union = area_r + area_c - inter
        iou = inter / jnp.maximum(union, 1e-9)
        return (iou > _IOU_THRESHOLD).astype(jnp.float32)

    ri = jax.lax.broadcasted_iota(jnp.int32, (_B, _B), 0)
    ci = jax.lax.broadcasted_iota(jnp.int32, (_B, _B), 1)
    tri = (ci > ri).astype(jnp.float32)
    ov_bb = over_tile(bi) * tri
    wp = wpack_ref[...]
    dn = (((1,), (0,)), ((), ()))
    packed_rows = jax.lax.dot_general(
        ov_bb.astype(jnp.bfloat16), wp, dn,
        preferred_element_type=jnp.float32).astype(jnp.int32)
    kr = ks[pl.ds(bi, 1), :]
    packed_kr = jax.lax.dot_general(
        kr.astype(jnp.bfloat16), wp, dn,
        preferred_element_type=jnp.float32).astype(jnp.int32)
    pk_vmem[pl.ds(0, _B), :] = packed_rows
    pk_vmem[pl.ds(_B, 1), :] = packed_kr
    copy = pltpu.make_async_copy(pk_vmem, pk_smem, sem)
    copy.start()
    copy.wait()

    words = [pk_smem[_B, w] for w in range(_W)]
    for j in range(_B):
        w = j // 16
        kj = (words[w] >> (j % 16)) & 1
        m = -kj
        for k in range(w, _W):
            words[k] = words[k] & ~(pk_smem[j, k] & m)

    lanei = jax.lax.broadcasted_iota(jnp.int32, (1, _B), 1)
    widx = lanei >> 4
    bidx = lanei & 15
    wsel = jnp.zeros((1, _B), jnp.int32)
    for w in range(_W):
        wsel = jnp.where(widx == w, words[w], wsel)
    kr_new = ((wsel >> bidx) & 1).astype(jnp.float32)
    ks[pl.ds(bi, 1), :] = kr_new
    out_ref[...] = kr_new.reshape(1, 1, _B)

    krb = kr_new.astype(jnp.bfloat16)

    def cross(i, c2):
        cb = bi + 1 + i

        @pl.when(cb < _T)
        def _():
            ov = over_tile(cb).astype(jnp.bfloat16)
            cnt = jax.lax.dot_general(
                krb, ov, dn, preferred_element_type=jnp.float32)
            kcb = ks[pl.ds(cb, 1), :]
            ks[pl.ds(cb, 1), :] = jnp.where(cnt > 0.0, 0.0, kcb)

        return c2

    jax.lax.fori_loop(0, _T - 1, cross, 0, unroll=2)


@functools.partial(jax.jit, static_argnames=("interpret",))
def _nms_pallas(keep0, boxes_p, x1c, y1c, x2c, y2c, wpack, interpret=False):
    full2 = lambda shape: pl.BlockSpec(shape, lambda bi: (0, 0))
    return pl.pallas_call(
        _nms_body,
        grid=(_T,),
        in_specs=[
            full2((_T, _B)),
            full2((_NP, 4)),
            full2((_T, _B)), full2((_T, _B)), full2((_T, _B)), full2((_T, _B)),
            full2((_B, _W)),
        ],
        out_specs=pl.BlockSpec((1, 1, _B), lambda bi: (bi, 0, 0)),
        out_shape=jax.ShapeDtypeStruct((_T, 1, _B), jnp.float32),
        scratch_shapes=[pltpu.VMEM((_T, _B), jnp.float32),
                        pltpu.VMEM((_B + 1, _W), jnp.int32),
                        pltpu.SMEM((_B + 1, _W), jnp.int32),
                        pltpu.SemaphoreType.DMA],
        interpret=interpret,
    )(keep0, boxes_p, x1c, y1c, x2c, y2c, wpack)


_l = np.arange(_B)
_wpack_np = np.zeros((_B, _W), np.float32)
_wpack_np[_l, _l // 16] = 2.0 ** (_l % 16)


def _run(boxes, scores, interpret=False):
    valid = scores > _SCORE_THRESHOLD
    eff = jnp.where(valid, scores, -1.0)
    neg_s, order = jax.lax.sort((-eff, jnp.arange(_N, dtype=jnp.int32)),
                                num_keys=1)
    b = boxes[order]
    s = -neg_s
    pad = _NP - _N
    b_p = jnp.pad(b, ((0, pad), (0, 0)))
    s_p = jnp.pad(s, (0, pad), constant_values=-1.0)
    keep0 = (s_p > 0.0).astype(jnp.float32).reshape(_T, _B)
    x1c = b_p[:, 0].reshape(_T, _B)
    y1c = b_p[:, 1].reshape(_T, _B)
    x2c = b_p[:, 2].reshape(_T, _B)
    y2c = b_p[:, 3].reshape(_T, _B)
    wpack = jnp.asarray(_wpack_np, jnp.bfloat16)
    keep = _nms_pallas(keep0, b_p, x1c, y1c, x2c, y2c, wpack,
                       interpret=interpret)
    keep_s = keep.reshape(_NP)[:_N] > 0.0
    kept_scores_sorted = jnp.maximum(s * keep_s.astype(jnp.float32), 0.0)
    out_scores = jnp.zeros((_N,), jnp.float32).at[order].set(kept_scores_sorted)
    keep_mask = out_scores > 0.0
    return out_scores, keep_mask


def kernel(boxes, scores):
    return _run(boxes, scores)
